# baseline (speedup 1.0000x reference)
import functools
import math

import jax
import jax.numpy as jnp
from jax import lax
from jax.experimental import pallas as pl
from jax.experimental.pallas import tpu as pltpu
from jax.experimental.pallas import tpu_sc as plsc

_OMEGA = 10.0
_EPSILON = 2.0
_VAR0 = 0.1
_VAR1 = 0.2
_THRESHOLD = 0.35
_ALPHA = 0.25
_GAMMA = 2.0
_WING_C = _OMEGA - _OMEGA * math.log(1.0 + _OMEGA / _EPSILON)

_P = 16800
_PV = _P // 16
_PP = 17408
_R = 8
_C = 2176
_G = 64
_B = 32
_L = 16

_PERM = [0, 1, 2, 3] + list(range(4, 18, 2)) + list(range(5, 18, 2))


@functools.partial(
    pl.kernel,
    out_type=(
        jax.ShapeDtypeStruct((_B, _P), jnp.float32),
        jax.ShapeDtypeStruct((_B, _P), jnp.int32),
        jax.ShapeDtypeStruct((_B, _G * _L), jnp.float32),
        jax.ShapeDtypeStruct((_B, _G * _L), jnp.int32),
    ),
    mesh=plsc.VectorSubcoreMesh(core_axis_name="c", subcore_axis_name="s",
                                num_cores=2, num_subcores=16),
    scratch_types=[
        pltpu.VMEM((5 * _P,), jnp.float32),
        pltpu.VMEM((_G * 5 * _L,), jnp.float32),
        pltpu.VMEM((_P,), jnp.float32),
        pltpu.VMEM((_P,), jnp.int32),
        pltpu.VMEM((_G * _L,), jnp.float32),
        pltpu.VMEM((_G * _L,), jnp.int32),
    ],
    compiler_params=pltpu.CompilerParams(use_tc_tiling_on_sc=False),
)
def _match_sc(pri_hbm, tt_hbm, bto_hbm, bti_hbm, vg_hbm, vig_hbm,
              pri_s, tt_s, bto_s, bti_s, vg_s, vig_s):
    wid = lax.axis_index("s") * 2 + lax.axis_index("c")

    pltpu.sync_copy(pri_hbm, pri_s)
    pltpu.sync_copy(tt_hbm.at[wid], tt_s)

    neg1 = jnp.full((_L,), -1.0, jnp.float32)
    zero_i = jnp.zeros((_L,), jnp.int32)
    for g in range(_G):
        vg_s[pl.ds(g * _L, _L)] = neg1
        vig_s[pl.ds(g * _L, _L)] = zero_i

    lane = lax.iota(jnp.int32, _L)

    def body(v, carry):
        s0 = v * _L
        pxa1 = pri_s[pl.ds(s0, _L)]
        pya1 = pri_s[pl.ds(_P + s0, _L)]
        pxa2 = pri_s[pl.ds(2 * _P + s0, _L)]
        pya2 = pri_s[pl.ds(3 * _P + s0, _L)]
        paa = pri_s[pl.ds(4 * _P + s0, _L)]
        pidxa = s0 + lane

        btoa = jnp.full((_L,), -1.0, jnp.float32)
        btia = jnp.zeros((_L,), jnp.int32)
        for g in range(_G):
            tx1 = tt_s[pl.ds(g * 80, _L)]
            ty1 = tt_s[pl.ds(g * 80 + _L, _L)]
            tx2 = tt_s[pl.ds(g * 80 + 2 * _L, _L)]
            ty2 = tt_s[pl.ds(g * 80 + 3 * _L, _L)]
            ta = tt_s[pl.ds(g * 80 + 4 * _L, _L)]
            ixa = jnp.maximum(
                jnp.minimum(tx2, pxa2) - jnp.maximum(tx1, pxa1), 0.0)
            iya = jnp.maximum(
                jnp.minimum(ty2, pya2) - jnp.maximum(ty1, pya1), 0.0)
            intera = ixa * iya
            ioua = intera / ((ta + paa) - intera)
            ma = ioua > btoa
            btoa = jnp.where(ma, ioua, btoa)
            btia = jnp.where(ma, g, btia)
            vg = vg_s[pl.ds(g * _L, _L)]
            vi = vig_s[pl.ds(g * _L, _L)]
            m2 = ioua > vg
            vg_s[pl.ds(g * _L, _L)] = jnp.where(m2, ioua, vg)
            vig_s[pl.ds(g * _L, _L)] = jnp.where(m2, pidxa, vi)
        bto_s[pl.ds(s0, _L)] = btoa
        bti_s[pl.ds(s0, _L)] = btia
        return carry

    lax.fori_loop(0, _PV, body, 0)

    pltpu.sync_copy(bto_s, bto_hbm.at[wid])
    pltpu.sync_copy(bti_s, bti_hbm.at[wid])
    pltpu.sync_copy(vg_s, vg_hbm.at[wid])
    pltpu.sync_copy(vig_s, vig_hbm.at[wid])


def _loss_kernel(conf_ref, regt_ref, priors_ref, tgt_ref, bto_ref, bti_ref,
                 vg_ref, vig_ref, out_ref):
    tgt = tgt_ref[0]

    pcx = priors_ref[0:1, :]
    pcy = priors_ref[1:2, :]
    pw = priors_ref[2:3, :]
    ph = priors_ref[3:4, :]

    pidx = lax.broadcasted_iota(jnp.int32, (1, _PP), 1)
    gidx = lax.broadcasted_iota(jnp.int32, (_G, 1), 0)

    bto = bto_ref[0]
    bti = bti_ref[0]

    vg = vg_ref[0]
    vig = vig_ref[0]
    vmax = jnp.max(vg, axis=1, keepdims=True)
    bpi = jnp.min(jnp.where(vg == vmax, vig, jnp.int32(1 << 30)),
                  axis=1, keepdims=True)

    eq = bpi == pidx
    forced_g = jnp.max(jnp.where(eq, gidx, -1), axis=0, keepdims=True)
    forced = forced_g >= 0
    bti = jnp.where(forced, forced_g, bti)
    bto = jnp.where(forced, 2.0, bto)

    onehot = (gidx == bti).astype(jnp.float32)
    matched = lax.dot_general(
        tgt, onehot, (((0,), (0,)), ((), ())),
        preferred_element_type=jnp.float32)

    lab = matched[18:19, :]
    conf = jnp.where(bto < _THRESHOLD, 0.0, lab)
    mpos = (conf != 0.0).astype(jnp.float32)
    mpos1 = (conf > 0.0).astype(jnp.float32)

    rw = 1.0 / pw
    rh = 1.0 / ph
    wrx = (1.0 / _VAR0) * rw
    wry = (1.0 / _VAR0) * rh

    mx1 = matched[0:1, :]
    my1 = matched[1:2, :]
    mx2 = matched[2:3, :]
    my2 = matched[3:4, :]
    g_cx = ((mx1 + mx2) * 0.5 - pcx) * wrx
    g_cy = ((my1 + my2) * 0.5 - pcy) * wry
    g_w = jnp.log((mx2 - mx1) * rw) * (1.0 / _VAR1)
    g_h = jnp.log((my2 - my1) * rh) * (1.0 / _VAR1)
    loc_t = jnp.concatenate([g_cx, g_cy, g_w, g_h], axis=0)

    x = regt_ref[0, 0:4, :] * (1.0 / 192.0)
    e = jnp.exp(-x)
    sig = 1.0 / (1.0 + e)
    bce = jnp.log1p(e) + (1.0 - loc_t) * x
    dqf = loc_t - sig
    qfl = dqf * dqf * bce
    qfl_sum = jnp.sum(qfl * mpos)
    n_pos = jnp.sum(mpos)

    lmd = regt_ref[0, 4:18, :] * (1.0 / 192.0)
    lmtx = (matched[4:11, :] - pcx) * wrx
    lmty = (matched[11:18, :] - pcy) * wry
    lm_t = jnp.concatenate([lmtx, lmty], axis=0)
    d = jnp.abs(lm_t - lmd)
    wing = jnp.where(d < _OMEGA, _OMEGA * jnp.log1p(d * (1.0 / _EPSILON)),
                     d - _WING_C)
    wing_sum = jnp.sum(wing * mpos1)
    n_pos1 = jnp.sum(mpos1)

    c8 = conf_ref[0]
    flat8 = (lax.broadcasted_iota(jnp.int32, (_R, _C), 0) * _C
             + lax.broadcasted_iota(jnp.int32, (_R, _C), 1))
    valid8 = flat8 < _P
    e8 = jnp.exp(-c8)
    lg8 = jnp.log1p(e8)
    y8 = 1.0 / (1.0 + e8)
    a8 = ((1.0 - _ALPHA) * _GAMMA) * (1.0 - y8) * lg8
    b8 = _ALPHA * y8 * y8 * (c8 + lg8)
    mpos8 = mpos.reshape(_R, _C)
    fl_sum = (jnp.sum(jnp.where(valid8, b8, 0.0))
              + jnp.sum(mpos8 * (a8 - b8)))

    b = pl.program_id(0)

    @pl.when(b == 0)
    def _init():
        for i in range(5):
            out_ref[i] = 0.0

    out_ref[0] += qfl_sum
    out_ref[1] += n_pos
    out_ref[2] += wing_sum
    out_ref[3] += n_pos1
    out_ref[4] += fl_sum


@jax.jit
def kernel(conf_data, reg_data, priors, targets):
    B, P, _ = conf_data.shape
    pad = _PP - P

    pcx = priors[:, 0]
    pcy = priors[:, 1]
    pw = priors[:, 2]
    ph = priors[:, 3]
    px1 = pcx - pw * 0.5
    py1 = pcy - ph * 0.5
    px2 = pcx + pw * 0.5
    py2 = pcy + ph * 0.5
    parea = (px2 - px1) * (py2 - py1)
    pri5 = jnp.stack(
        [px1, py1, px2, py2, parea], axis=0).reshape(5 * _P)

    tb = targets[:, :, 0:4]
    tarea = (tb[:, :, 2] - tb[:, :, 0]) * (tb[:, :, 3] - tb[:, :, 1])
    tt5 = jnp.concatenate([tb, tarea[:, :, None]], axis=2)
    ttb = jnp.broadcast_to(
        tt5[:, :, :, None], (B, _G, 5, _L)).reshape(B, _G * 5 * _L)

    bto, bti, vg, vig = _match_sc(pri5, ttb)
    vg = vg.reshape(B, _G, _L)
    vig = vig.reshape(B, _G, _L)

    bto_p = jnp.pad(bto, ((0, 0), (0, pad)))[:, None, :]
    bti_p = jnp.pad(bti, ((0, 0), (0, pad)))[:, None, :]
    vg_p = jnp.pad(vg, ((0, 0), (0, 0), (0, 128 - _L)), constant_values=-1.0)
    vig_p = jnp.pad(vig, ((0, 0), (0, 0), (0, 128 - _L)),
                    constant_values=1 << 30)

    conf_p = jnp.pad(conf_data[:, :, 0], ((0, 0), (0, pad)))
    conf_p = conf_p.reshape(B, _R, _C)
    regt = jnp.transpose(reg_data, (0, 2, 1))[:, _PERM, :]
    regt_p = jnp.pad(regt, ((0, 0), (0, 0), (0, pad)))
    pt = jnp.transpose(priors, (1, 0))
    padvals = jnp.concatenate(
        [jnp.full((2, pad), -10.0, jnp.float32),
         jnp.ones((2, pad), jnp.float32)], axis=0)
    priors_p = jnp.concatenate([pt, padvals], axis=1)
    tgt_p = targets[:, :, _PERM + [18]]

    sums = pl.pallas_call(
        _loss_kernel,
        grid=(B,),
        in_specs=[
            pl.BlockSpec((1, _R, _C), lambda b: (b, 0, 0)),
            pl.BlockSpec((1, 18, _PP), lambda b: (b, 0, 0)),
            pl.BlockSpec((4, _PP), lambda b: (0, 0)),
            pl.BlockSpec((1, _G, 19), lambda b: (b, 0, 0)),
            pl.BlockSpec((1, 1, _PP), lambda b: (b, 0, 0)),
            pl.BlockSpec((1, 1, _PP), lambda b: (b, 0, 0)),
            pl.BlockSpec((1, _G, 128), lambda b: (b, 0, 0)),
            pl.BlockSpec((1, _G, 128), lambda b: (b, 0, 0)),
        ],
        out_specs=pl.BlockSpec(memory_space=pltpu.SMEM),
        out_shape=jax.ShapeDtypeStruct((5,), jnp.float32),
        compiler_params=pltpu.CompilerParams(
            dimension_semantics=("arbitrary",)),
    )(conf_p, regt_p, priors_p, tgt_p, bto_p, bti_p, vg_p, vig_p)

    qfl_sum, n_pos, wing_sum, n_pos1, fl_sum = (
        sums[0], sums[1], sums[2], sums[3], sums[4])
    loss_l = qfl_sum / jnp.maximum(n_pos * 4.0, 1.0)
    loss_landm = wing_sum / jnp.maximum(n_pos1 * 14.0, 1.0)
    loss_c = fl_sum / (B * P)
    return (loss_l, loss_c, loss_landm)


# speedup vs baseline: 1.0803x; 1.0803x over previous
import functools
import math

import jax
import jax.numpy as jnp
from jax import lax
from jax.experimental import pallas as pl
from jax.experimental.pallas import tpu as pltpu
from jax.experimental.pallas import tpu_sc as plsc

_OMEGA = 10.0
_EPSILON = 2.0
_VAR0 = 0.1
_VAR1 = 0.2
_THRESHOLD = 0.35
_ALPHA = 0.25
_GAMMA = 2.0
_WING_C = _OMEGA - _OMEGA * math.log(1.0 + _OMEGA / _EPSILON)

_P = 16800
_PV = _P // 16
_PP = 17408
_R = 8
_C = 2176
_G = 64
_B = 32
_L = 16

_PERM = [0, 1, 2, 3] + list(range(4, 18, 2)) + list(range(5, 18, 2))


@functools.partial(
    pl.kernel,
    out_type=(
        jax.ShapeDtypeStruct((_B, _P), jnp.float32),
        jax.ShapeDtypeStruct((_B, _P), jnp.int32),
        jax.ShapeDtypeStruct((_B, _G * _L), jnp.float32),
        jax.ShapeDtypeStruct((_B, _G * _L), jnp.int32),
    ),
    mesh=plsc.VectorSubcoreMesh(core_axis_name="c", subcore_axis_name="s",
                                num_cores=2, num_subcores=16),
    scratch_types=[
        pltpu.VMEM((5 * _P,), jnp.float32),
        pltpu.VMEM((_G * 5 * _L,), jnp.float32),
        pltpu.VMEM((_P,), jnp.float32),
        pltpu.VMEM((_P,), jnp.int32),
        pltpu.VMEM((_G * _L,), jnp.float32),
        pltpu.VMEM((_G * _L,), jnp.int32),
    ],
    compiler_params=pltpu.CompilerParams(use_tc_tiling_on_sc=False),
)
def _match_sc(pri_hbm, tt_hbm, bto_hbm, bti_hbm, vg_hbm, vig_hbm,
              pri_s, tt_s, bto_s, bti_s, vg_s, vig_s):
    wid = lax.axis_index("s") * 2 + lax.axis_index("c")

    pltpu.sync_copy(pri_hbm, pri_s)
    pltpu.sync_copy(tt_hbm.at[wid], tt_s)

    neg1 = jnp.full((_L,), -1.0, jnp.float32)
    zero_i = jnp.zeros((_L,), jnp.int32)
    for g in range(_G):
        vg_s[pl.ds(g * _L, _L)] = neg1
        vig_s[pl.ds(g * _L, _L)] = zero_i

    lane = lax.iota(jnp.int32, _L)

    def body(v, carry):
        s0 = v * (2 * _L)
        s1 = s0 + _L
        pxa1 = pri_s[pl.ds(s0, _L)]
        pya1 = pri_s[pl.ds(_P + s0, _L)]
        pxa2 = pri_s[pl.ds(2 * _P + s0, _L)]
        pya2 = pri_s[pl.ds(3 * _P + s0, _L)]
        paa = pri_s[pl.ds(4 * _P + s0, _L)]
        pxb1 = pri_s[pl.ds(s1, _L)]
        pyb1 = pri_s[pl.ds(_P + s1, _L)]
        pxb2 = pri_s[pl.ds(2 * _P + s1, _L)]
        pyb2 = pri_s[pl.ds(3 * _P + s1, _L)]
        pab = pri_s[pl.ds(4 * _P + s1, _L)]
        pidxa = s0 + lane
        pidxb = s1 + lane

        btoa = jnp.full((_L,), -1.0, jnp.float32)
        btia = jnp.zeros((_L,), jnp.int32)
        btob = jnp.full((_L,), -1.0, jnp.float32)
        btib = jnp.zeros((_L,), jnp.int32)
        for g in range(_G):
            tx1 = tt_s[pl.ds(g * 80, _L)]
            ty1 = tt_s[pl.ds(g * 80 + _L, _L)]
            tx2 = tt_s[pl.ds(g * 80 + 2 * _L, _L)]
            ty2 = tt_s[pl.ds(g * 80 + 3 * _L, _L)]
            ta = tt_s[pl.ds(g * 80 + 4 * _L, _L)]
            ixa = jnp.maximum(
                jnp.minimum(tx2, pxa2) - jnp.maximum(tx1, pxa1), 0.0)
            iya = jnp.maximum(
                jnp.minimum(ty2, pya2) - jnp.maximum(ty1, pya1), 0.0)
            intera = ixa * iya
            ioua = intera / ((ta + paa) - intera)
            ixb = jnp.maximum(
                jnp.minimum(tx2, pxb2) - jnp.maximum(tx1, pxb1), 0.0)
            iyb = jnp.maximum(
                jnp.minimum(ty2, pyb2) - jnp.maximum(ty1, pyb1), 0.0)
            interb = ixb * iyb
            ioub = interb / ((ta + pab) - interb)
            ma = ioua > btoa
            btoa = jnp.where(ma, ioua, btoa)
            btia = jnp.where(ma, g, btia)
            mb = ioub > btob
            btob = jnp.where(mb, ioub, btob)
            btib = jnp.where(mb, g, btib)
            vg = vg_s[pl.ds(g * _L, _L)]
            vi = vig_s[pl.ds(g * _L, _L)]
            m2a = ioua > vg
            vg1 = jnp.where(m2a, ioua, vg)
            vi1 = jnp.where(m2a, pidxa, vi)
            m2b = ioub > vg1
            vg_s[pl.ds(g * _L, _L)] = jnp.where(m2b, ioub, vg1)
            vig_s[pl.ds(g * _L, _L)] = jnp.where(m2b, pidxb, vi1)
        bto_s[pl.ds(s0, _L)] = btoa
        bti_s[pl.ds(s0, _L)] = btia
        bto_s[pl.ds(s1, _L)] = btob
        bti_s[pl.ds(s1, _L)] = btib
        return carry

    lax.fori_loop(0, _PV // 2, body, 0)

    pltpu.sync_copy(bto_s, bto_hbm.at[wid])
    pltpu.sync_copy(bti_s, bti_hbm.at[wid])
    pltpu.sync_copy(vg_s, vg_hbm.at[wid])
    pltpu.sync_copy(vig_s, vig_hbm.at[wid])


def _loss_kernel(conf_ref, regt_ref, priors_ref, tgt_ref, bto_ref, bti_ref,
                 vg_ref, vig_ref, out_ref):
    tgt = tgt_ref[0]

    pcx = priors_ref[0:1, :]
    pcy = priors_ref[1:2, :]
    pw = priors_ref[2:3, :]
    ph = priors_ref[3:4, :]

    pidx = lax.broadcasted_iota(jnp.int32, (1, _PP), 1)
    gidx = lax.broadcasted_iota(jnp.int32, (_G, 1), 0)

    bto = bto_ref[0]
    bti = bti_ref[0]

    vg = vg_ref[0]
    vig = vig_ref[0]
    vmax = jnp.max(vg, axis=1, keepdims=True)
    bpi = jnp.min(jnp.where(vg == vmax, vig, jnp.int32(1 << 30)),
                  axis=1, keepdims=True)

    eq = bpi == pidx
    forced_g = jnp.max(jnp.where(eq, gidx, -1), axis=0, keepdims=True)
    forced = forced_g >= 0
    bti = jnp.where(forced, forced_g, bti)
    bto = jnp.where(forced, 2.0, bto)

    onehot = (gidx == bti).astype(jnp.float32)
    matched = lax.dot_general(
        tgt, onehot, (((0,), (0,)), ((), ())),
        preferred_element_type=jnp.float32)

    lab = matched[18:19, :]
    conf = jnp.where(bto < _THRESHOLD, 0.0, lab)
    mpos = (conf != 0.0).astype(jnp.float32)
    mpos1 = (conf > 0.0).astype(jnp.float32)

    rw = 1.0 / pw
    rh = 1.0 / ph
    wrx = (1.0 / _VAR0) * rw
    wry = (1.0 / _VAR0) * rh

    mx1 = matched[0:1, :]
    my1 = matched[1:2, :]
    mx2 = matched[2:3, :]
    my2 = matched[3:4, :]
    g_cx = ((mx1 + mx2) * 0.5 - pcx) * wrx
    g_cy = ((my1 + my2) * 0.5 - pcy) * wry
    g_w = jnp.log((mx2 - mx1) * rw) * (1.0 / _VAR1)
    g_h = jnp.log((my2 - my1) * rh) * (1.0 / _VAR1)
    loc_t = jnp.concatenate([g_cx, g_cy, g_w, g_h], axis=0)

    x = regt_ref[0, 0:4, :] * (1.0 / 192.0)
    e = jnp.exp(-x)
    sig = 1.0 / (1.0 + e)
    bce = jnp.log1p(e) + (1.0 - loc_t) * x
    dqf = loc_t - sig
    qfl = dqf * dqf * bce
    qfl_sum = jnp.sum(qfl * mpos)
    n_pos = jnp.sum(mpos)

    lmd = regt_ref[0, 4:18, :] * (1.0 / 192.0)
    lmtx = (matched[4:11, :] - pcx) * wrx
    lmty = (matched[11:18, :] - pcy) * wry
    lm_t = jnp.concatenate([lmtx, lmty], axis=0)
    d = jnp.abs(lm_t - lmd)
    wing = jnp.where(d < _OMEGA, _OMEGA * jnp.log1p(d * (1.0 / _EPSILON)),
                     d - _WING_C)
    wing_sum = jnp.sum(wing * mpos1)
    n_pos1 = jnp.sum(mpos1)

    c8 = conf_ref[0]
    flat8 = (lax.broadcasted_iota(jnp.int32, (_R, _C), 0) * _C
             + lax.broadcasted_iota(jnp.int32, (_R, _C), 1))
    valid8 = flat8 < _P
    e8 = jnp.exp(-c8)
    lg8 = jnp.log1p(e8)
    y8 = 1.0 / (1.0 + e8)
    a8 = ((1.0 - _ALPHA) * _GAMMA) * (1.0 - y8) * lg8
    b8 = _ALPHA * y8 * y8 * (c8 + lg8)
    mpos8 = mpos.reshape(_R, _C)
    fl_sum = (jnp.sum(jnp.where(valid8, b8, 0.0))
              + jnp.sum(mpos8 * (a8 - b8)))

    b = pl.program_id(0)

    @pl.when(b == 0)
    def _init():
        for i in range(5):
            out_ref[i] = 0.0

    out_ref[0] += qfl_sum
    out_ref[1] += n_pos
    out_ref[2] += wing_sum
    out_ref[3] += n_pos1
    out_ref[4] += fl_sum


@jax.jit
def kernel(conf_data, reg_data, priors, targets):
    B, P, _ = conf_data.shape
    pad = _PP - P

    pcx = priors[:, 0]
    pcy = priors[:, 1]
    pw = priors[:, 2]
    ph = priors[:, 3]
    px1 = pcx - pw * 0.5
    py1 = pcy - ph * 0.5
    px2 = pcx + pw * 0.5
    py2 = pcy + ph * 0.5
    parea = (px2 - px1) * (py2 - py1)
    pri5 = jnp.stack(
        [px1, py1, px2, py2, parea], axis=0).reshape(5 * _P)

    tb = targets[:, :, 0:4]
    tarea = (tb[:, :, 2] - tb[:, :, 0]) * (tb[:, :, 3] - tb[:, :, 1])
    tt5 = jnp.concatenate([tb, tarea[:, :, None]], axis=2)
    ttb = jnp.broadcast_to(
        tt5[:, :, :, None], (B, _G, 5, _L)).reshape(B, _G * 5 * _L)

    bto, bti, vg, vig = _match_sc(pri5, ttb)
    vg = vg.reshape(B, _G, _L)
    vig = vig.reshape(B, _G, _L)

    bto_p = jnp.pad(bto, ((0, 0), (0, pad)))[:, None, :]
    bti_p = jnp.pad(bti, ((0, 0), (0, pad)))[:, None, :]
    vg_p = jnp.pad(vg, ((0, 0), (0, 0), (0, 128 - _L)), constant_values=-1.0)
    vig_p = jnp.pad(vig, ((0, 0), (0, 0), (0, 128 - _L)),
                    constant_values=1 << 30)

    conf_p = jnp.pad(conf_data[:, :, 0], ((0, 0), (0, pad)))
    conf_p = conf_p.reshape(B, _R, _C)
    regt = jnp.transpose(reg_data, (0, 2, 1))[:, _PERM, :]
    regt_p = jnp.pad(regt, ((0, 0), (0, 0), (0, pad)))
    pt = jnp.transpose(priors, (1, 0))
    padvals = jnp.concatenate(
        [jnp.full((2, pad), -10.0, jnp.float32),
         jnp.ones((2, pad), jnp.float32)], axis=0)
    priors_p = jnp.concatenate([pt, padvals], axis=1)
    tgt_p = targets[:, :, _PERM + [18]]

    sums = pl.pallas_call(
        _loss_kernel,
        grid=(B,),
        in_specs=[
            pl.BlockSpec((1, _R, _C), lambda b: (b, 0, 0)),
            pl.BlockSpec((1, 18, _PP), lambda b: (b, 0, 0)),
            pl.BlockSpec((4, _PP), lambda b: (0, 0)),
            pl.BlockSpec((1, _G, 19), lambda b: (b, 0, 0)),
            pl.BlockSpec((1, 1, _PP), lambda b: (b, 0, 0)),
            pl.BlockSpec((1, 1, _PP), lambda b: (b, 0, 0)),
            pl.BlockSpec((1, _G, 128), lambda b: (b, 0, 0)),
            pl.BlockSpec((1, _G, 128), lambda b: (b, 0, 0)),
        ],
        out_specs=pl.BlockSpec(memory_space=pltpu.SMEM),
        out_shape=jax.ShapeDtypeStruct((5,), jnp.float32),
        compiler_params=pltpu.CompilerParams(
            dimension_semantics=("arbitrary",)),
    )(conf_p, regt_p, priors_p, tgt_p, bto_p, bti_p, vg_p, vig_p)

    qfl_sum, n_pos, wing_sum, n_pos1, fl_sum = (
        sums[0], sums[1], sums[2], sums[3], sums[4])
    loss_l = qfl_sum / jnp.maximum(n_pos * 4.0, 1.0)
    loss_landm = wing_sum / jnp.maximum(n_pos1 * 14.0, 1.0)
    loss_c = fl_sum / (B * P)
    return (loss_l, loss_c, loss_landm)
